# Initial kernel scaffold; baseline (speedup 1.0000x reference)
import jax
import jax.numpy as jnp
from jax.experimental import pallas as pl


def kernel(node_fea, edge_fea, edge_sh, edge_length_embedding, edge_vec, D, params, edge_src, edge_dst, batch):
    raise NotImplementedError("write your pallas kernel here")


# speedup vs baseline: 8.4417x; 8.4417x over previous
import functools
import math

import jax
import jax.numpy as jnp
from jax import lax
from jax.experimental import pallas as pl
from jax.experimental.pallas import tpu as pltpu

SC = 128
H = 8
QK = 16
HD = 32
F32 = jnp.float32


def _ln(x, g, b, eps=1e-6):
    m = jnp.mean(x, axis=-1, keepdims=True)
    v = jnp.mean((x - m) ** 2, axis=-1, keepdims=True)
    return (x - m) / jnp.sqrt(v + eps) * g + b


def _silu(x):
    return x * jax.nn.sigmoid(x)


def _dot(a, b):
    return jax.lax.dot(a, b, preferred_element_type=F32)


def _node_prep_body(x_ref, lng, lnb, qw1, qb1, qg, qbn, qw2, qb2,
                    kw1, kb1, kg, kbn, kw2, kb2, ws, wd, nsw, nsb,
                    ts_ref, td_ref, ns_ref):
    x = x_ref[...]
    node = _ln(x, lng[...], lnb[...])
    scal = node[:, :SC]

    def qkproj(w1, b1, g, bn, w2, b2):
        h = _silu(_ln(_dot(scal, w1[...]) + b1[...], g[...], bn[...]))
        return _dot(h, w2[...]) + b2[...]

    qn = qkproj(qw1, qb1, qg, qbn, qw2, qb2)
    kn = qkproj(kw1, kb1, kg, kbn, kw2, kb2)
    ts_ref[:, :256] = _dot(node, ws[...])
    ts_ref[:, 256:] = kn
    td_ref[:, :256] = _dot(node, wd[...])
    td_ref[:, 256:] = qn
    ns_ref[...] = _dot(x, nsw[...]) + nsb[...]


def _node_prep(node_fea, p, bn=1000):
    n, dn = node_fea.shape
    grid = n // bn

    def row_spec(d):
        return pl.BlockSpec((bn, d), lambda i: (i, 0))

    def w_spec(a):
        return pl.BlockSpec(a.shape, lambda i: tuple(0 for _ in a.shape))

    r2 = lambda a: a.reshape(1, -1)
    weights = [r2(p["ln_node_g"]), r2(p["ln_node_b"]),
               p["q_w1"], r2(p["q_b1"]), r2(p["q_g"]), r2(p["q_bn"]),
               p["q_w2"], r2(p["q_b2"]),
               p["k_w1"], r2(p["k_b1"]), r2(p["k_g"]), r2(p["k_bn"]),
               p["k_w2"], r2(p["k_b2"]),
               p["pre_w"][:dn], p["pre_w"][dn:2 * dn],
               p["ns_w"], r2(p["ns_b"])]
    return pl.pallas_call(
        _node_prep_body,
        grid=(grid,),
        in_specs=[row_spec(dn)] + [w_spec(a) for a in weights],
        out_specs=[row_spec(384), row_spec(384), row_spec(dn)],
        out_shape=[jax.ShapeDtypeStruct((n, 384), F32),
                   jax.ShapeDtypeStruct((n, 384), F32),
                   jax.ShapeDtypeStruct((n, dn), F32)],
    )(node_fea, *weights)


def _edge_body(ef_ref, el_ref, dm_ref, ev_ref, gs_ref, gd_ref,
               lneg, lneb, aw1, ab1, ag1, abg1, aw2, ab2, ag2, abg2, aw3, ab3,
               we, preb, c1r1, c1r2, c1w, c1d, c1g, c1b,
               c2r1, c2r2, c2w, c2d, onw, onb, linew, lineb, esw, esb,
               sel, expm, emsg_ref, eout_ref):
    ef = ef_ref[...]
    el = el_ref[...]
    edge = _ln(ef, lneg[...], lneb[...])
    h = _silu(_ln(_dot(el, aw1[...]) + ab1[...], ag1[...], abg1[...]))
    h = _silu(_ln(_dot(h, aw2[...]) + ab2[...], ag2[...], abg2[...]))
    bias = _dot(h, aw3[...]) + ab3[...]
    gs = gs_ref[...]
    gd = gd_ref[...]
    qk = _dot(gs[:, 256:] * gd[:, 256:], sel[...]) * (1.0 / math.sqrt(QK))
    alpha = qk + bias
    msg = gs[:, :256] + gd[:, :256] + _dot(edge, we[...]) + preb[...]
    dm = dm_ref[...]
    r1 = _dot(_silu(_dot(el, c1r1[...])), c1r2[...])
    v = _dot(msg * r1, c1w[...]) + _dot(dm, c1d[...])
    v = _silu(_ln(v, c1g[...], c1b[...]))
    r2 = _dot(_silu(_dot(el, c2r1[...])), c2r2[...])
    value = _dot(v * r2, c2w[...]) + _dot(dm, c2d[...])
    av = _dot(alpha, expm[...])
    emsg_ref[...] = value * av
    ev = ev_ref[...]
    ons = (ev[:, 0:1] * ev[:, 0:1] + ev[:, 1:2] * ev[:, 1:2]
           + ev[:, 2:3] * ev[:, 2:3]) < 1e-20

    @pl.when(jnp.any(ons))
    def _():
        onsite_val = _dot(msg, onw[...]) + onb[...]
        emsg_ref[...] = jnp.where(ons, onsite_val, value) * av

    eout_ref[...] = (_dot(emsg_ref[...], linew[...]) + lineb[...]
                     + _dot(ef, esw[...]) + esb[...])


def _edge_stage(edge_fea, elen, dm, edge_vec, gs, gd, p, be=1280):
    e, de = edge_fea.shape
    grid = e // be
    sel = (jnp.arange(SC)[:, None] // QK == jnp.arange(H)[None, :]).astype(F32)
    expm = (jnp.arange(H)[:, None] == jnp.arange(H * HD)[None, :] // HD).astype(F32)

    def row_spec(d):
        return pl.BlockSpec((be, d), lambda i: (i, 0))

    def w_spec(a):
        return pl.BlockSpec(a.shape, lambda i: tuple(0 for _ in a.shape))

    r2 = lambda a: a.reshape(1, -1)
    weights = [r2(p["ln_edge_g"]), r2(p["ln_edge_b"]),
               p["a_w1"], r2(p["a_b1"]), r2(p["a_g1"]), r2(p["a_bg1"]),
               p["a_w2"], r2(p["a_b2"]), r2(p["a_g2"]), r2(p["a_bg2"]),
               p["a_w3"], r2(p["a_b3"]),
               p["pre_w"][512:], r2(p["pre_b"]),
               p["c1_r1"], p["c1_r2"], p["c1_w"], p["c1_d"],
               r2(p["c1_g"]), r2(p["c1_b"]),
               p["c2_r1"], p["c2_r2"], p["c2_w"], p["c2_d"],
               p["on_w"], r2(p["on_b"]),
               p["line_w"], r2(p["line_b"]), p["es_w"], r2(p["es_b"]),
               sel, expm]
    return pl.pallas_call(
        _edge_body,
        grid=(grid,),
        in_specs=[row_spec(de), row_spec(64), row_spec(9), row_spec(3),
                  row_spec(384), row_spec(384)] + [w_spec(a) for a in weights],
        out_specs=[row_spec(256), row_spec(de)],
        out_shape=[jax.ShapeDtypeStruct((e, 256), F32),
                   jax.ShapeDtypeStruct((e, de), F32)],
    )(edge_fea, elen, dm, edge_vec, gs, gd, *weights)


def _node_out_body(nmsg_ref, ns_ref, linw, linb, out_ref):
    out_ref[...] = _dot(nmsg_ref[...], linw[...]) + linb[...] + ns_ref[...]


def _node_out(nmsg, ns, p, bn=1000):
    n, dn = ns.shape
    grid = n // bn

    def row_spec(d):
        return pl.BlockSpec((bn, d), lambda i: (i, 0))

    def w_spec(a):
        return pl.BlockSpec(a.shape, lambda i: tuple(0 for _ in a.shape))

    linb = p["lin_b"].reshape(1, -1)
    return pl.pallas_call(
        _node_out_body,
        grid=(grid,),
        in_specs=[row_spec(256), row_spec(dn), w_spec(p["lin_w"]), w_spec(linb)],
        out_specs=row_spec(dn),
        out_shape=jax.ShapeDtypeStruct((n, dn), F32),
    )(nmsg, ns, p["lin_w"], linb)


def kernel(node_fea, edge_fea, edge_sh, edge_length_embedding, edge_vec, D,
           params, edge_src, edge_dst, batch):
    p = params
    n = node_fea.shape[0]
    e = edge_fea.shape[0]
    table_s, table_d, ns = _node_prep(node_fea, p)
    gs = table_s[edge_src]
    gd = table_d[edge_dst]
    dm = D.reshape(e, 9)
    emsg, edge_out = _edge_stage(edge_fea, edge_length_embedding, dm,
                                 edge_vec, gs, gd, p)
    nmsg = jax.ops.segment_sum(emsg, edge_dst, num_segments=n)
    node_out = _node_out(nmsg, ns, p)
    return node_out, edge_out


# speedup vs baseline: 11.5281x; 1.3656x over previous
import functools
import math

import jax
import jax.numpy as jnp
from jax import lax
from jax.experimental import pallas as pl
from jax.experimental.pallas import tpu as pltpu
from jax.experimental.pallas import tpu_sc as plsc

SC = 128
H = 8
QK = 16
HD = 32
F32 = jnp.float32


def _ln(x, g, b, eps=1e-6):
    m = jnp.mean(x, axis=-1, keepdims=True)
    v = jnp.mean((x - m) ** 2, axis=-1, keepdims=True)
    return (x - m) / jnp.sqrt(v + eps) * g + b


def _silu(x):
    return x * jax.nn.sigmoid(x)


def _dot(a, b):
    return jax.lax.dot(a, b, preferred_element_type=F32)


def _node_prep_body(x_ref, lng, lnb, qw1, qb1, qg, qbn, qw2, qb2,
                    kw1, kb1, kg, kbn, kw2, kb2, ws, wd, nsw, nsb,
                    ts_ref, td_ref, ns_ref):
    x = x_ref[...]
    node = _ln(x, lng[...], lnb[...])
    scal = node[:, :SC]

    def qkproj(w1, b1, g, bn, w2, b2):
        h = _silu(_ln(_dot(scal, w1[...]) + b1[...], g[...], bn[...]))
        return _dot(h, w2[...]) + b2[...]

    qn = qkproj(qw1, qb1, qg, qbn, qw2, qb2)
    kn = qkproj(kw1, kb1, kg, kbn, kw2, kb2)
    ts_ref[:, :256] = _dot(node, ws[...])
    ts_ref[:, 256:] = kn
    td_ref[:, :256] = _dot(node, wd[...])
    td_ref[:, 256:] = qn
    ns_ref[...] = _dot(x, nsw[...]) + nsb[...]


def _node_prep(node_fea, p, bn=1000):
    n, dn = node_fea.shape
    grid = n // bn

    def row_spec(d):
        return pl.BlockSpec((bn, d), lambda i: (i, 0))

    def w_spec(a):
        return pl.BlockSpec(a.shape, lambda i: tuple(0 for _ in a.shape))

    r2 = lambda a: a.reshape(1, -1)
    weights = [r2(p["ln_node_g"]), r2(p["ln_node_b"]),
               p["q_w1"], r2(p["q_b1"]), r2(p["q_g"]), r2(p["q_bn"]),
               p["q_w2"], r2(p["q_b2"]),
               p["k_w1"], r2(p["k_b1"]), r2(p["k_g"]), r2(p["k_bn"]),
               p["k_w2"], r2(p["k_b2"]),
               p["pre_w"][:dn], p["pre_w"][dn:2 * dn],
               p["ns_w"], r2(p["ns_b"])]
    return pl.pallas_call(
        _node_prep_body,
        grid=(grid,),
        in_specs=[row_spec(dn)] + [w_spec(a) for a in weights],
        out_specs=[row_spec(384), row_spec(384), row_spec(dn)],
        out_shape=[jax.ShapeDtypeStruct((n, 384), F32),
                   jax.ShapeDtypeStruct((n, 384), F32),
                   jax.ShapeDtypeStruct((n, dn), F32)],
    )(node_fea, *weights)


def _edge_body(ef_ref, el_ref, dm_ref, ev_ref, gs_ref, gd_ref,
               lneg, lneb, aw1, ab1, ag1, abg1, aw2, ab2, ag2, abg2, aw3, ab3,
               we, preb, c1r1, c1r2, c1w, c1d, c1g, c1b,
               c2r1, c2r2, c2w, c2d, onw, onb, linew, lineb, esw, esb,
               sel, expm, emsg_ref, eout_ref):
    ef = ef_ref[...]
    el = el_ref[...]
    edge = _ln(ef, lneg[...], lneb[...])
    h = _silu(_ln(_dot(el, aw1[...]) + ab1[...], ag1[...], abg1[...]))
    h = _silu(_ln(_dot(h, aw2[...]) + ab2[...], ag2[...], abg2[...]))
    bias = _dot(h, aw3[...]) + ab3[...]
    gs = gs_ref[...]
    gd = gd_ref[...]
    qk = _dot(gs[:, 256:] * gd[:, 256:], sel[...]) * (1.0 / math.sqrt(QK))
    alpha = qk + bias
    msg = gs[:, :256] + gd[:, :256] + _dot(edge, we[...]) + preb[...]
    dm = dm_ref[...]
    r1 = _dot(_silu(_dot(el, c1r1[...])), c1r2[...])
    v = _dot(msg * r1, c1w[...]) + _dot(dm, c1d[...])
    v = _silu(_ln(v, c1g[...], c1b[...]))
    r2 = _dot(_silu(_dot(el, c2r1[...])), c2r2[...])
    value = _dot(v * r2, c2w[...]) + _dot(dm, c2d[...])
    av = _dot(alpha, expm[...])
    emsg_ref[...] = value * av
    ev = ev_ref[...]
    ons = (ev[:, 0:1] * ev[:, 0:1] + ev[:, 1:2] * ev[:, 1:2]
           + ev[:, 2:3] * ev[:, 2:3]) < 1e-20

    @pl.when(jnp.any(ons))
    def _():
        onsite_val = _dot(msg, onw[...]) + onb[...]
        emsg_ref[...] = jnp.where(ons, onsite_val, value) * av

    eout_ref[...] = (_dot(emsg_ref[...], linew[...]) + lineb[...]
                     + _dot(ef, esw[...]) + esb[...])


def _edge_stage(edge_fea, elen, dm, edge_vec, gs, gd, p, be=1280):
    e, de = edge_fea.shape
    grid = e // be
    sel = (jnp.arange(SC)[:, None] // QK == jnp.arange(H)[None, :]).astype(F32)
    expm = (jnp.arange(H)[:, None] == jnp.arange(H * HD)[None, :] // HD).astype(F32)

    def row_spec(d):
        return pl.BlockSpec((be, d), lambda i: (i, 0))

    def w_spec(a):
        return pl.BlockSpec(a.shape, lambda i: tuple(0 for _ in a.shape))

    r2 = lambda a: a.reshape(1, -1)
    weights = [r2(p["ln_edge_g"]), r2(p["ln_edge_b"]),
               p["a_w1"], r2(p["a_b1"]), r2(p["a_g1"]), r2(p["a_bg1"]),
               p["a_w2"], r2(p["a_b2"]), r2(p["a_g2"]), r2(p["a_bg2"]),
               p["a_w3"], r2(p["a_b3"]),
               p["pre_w"][512:], r2(p["pre_b"]),
               p["c1_r1"], p["c1_r2"], p["c1_w"], p["c1_d"],
               r2(p["c1_g"]), r2(p["c1_b"]),
               p["c2_r1"], p["c2_r2"], p["c2_w"], p["c2_d"],
               p["on_w"], r2(p["on_b"]),
               p["line_w"], r2(p["line_b"]), p["es_w"], r2(p["es_b"]),
               sel, expm]
    return pl.pallas_call(
        _edge_body,
        grid=(grid,),
        in_specs=[row_spec(de), row_spec(64), row_spec(9), row_spec(3),
                  row_spec(384), row_spec(384)] + [w_spec(a) for a in weights],
        out_specs=[row_spec(256), row_spec(de)],
        out_shape=[jax.ShapeDtypeStruct((e, 256), F32),
                   jax.ShapeDtypeStruct((e, de), F32)],
    )(edge_fea, elen, dm, edge_vec, gs, gd, *weights)


def _sc_gather(table_s, table_d, edge_src, edge_dst):
    e = edge_src.shape[0]
    d = table_s.shape[1]
    nw = 32
    per_w = e // nw
    c = 128
    n_chunks = per_w // c
    rem = per_w - n_chunks * c
    mesh = plsc.VectorSubcoreMesh(core_axis_name="c", subcore_axis_name="s")

    @functools.partial(
        pl.kernel, mesh=mesh,
        out_type=[jax.ShapeDtypeStruct((e, d), F32),
                  jax.ShapeDtypeStruct((e, d), F32)],
        scratch_types=[pltpu.VMEM((c,), jnp.int32), pltpu.VMEM((c,), jnp.int32),
                       pltpu.VMEM((c, d), F32), pltpu.VMEM((c, d), F32),
                       pltpu.SemaphoreType.DMA, pltpu.SemaphoreType.DMA],
    )
    def gk(ts_hbm, td_hbm, src_hbm, dst_hbm, gs_hbm, gd_hbm,
           idx_s, idx_d, rows_s, rows_d, sem1, sem2):
        wid = lax.axis_index("s") * 2 + lax.axis_index("c")
        base = wid * per_w

        def chunk(off, csz):
            i_s = idx_s.at[pl.ds(0, csz)] if csz != c else idx_s
            i_d = idx_d.at[pl.ds(0, csz)] if csz != c else idx_d
            r_s = rows_s.at[pl.ds(0, csz)] if csz != c else rows_s
            r_d = rows_d.at[pl.ds(0, csz)] if csz != c else rows_d
            pltpu.sync_copy(src_hbm.at[pl.ds(off, csz)], i_s)
            pltpu.sync_copy(dst_hbm.at[pl.ds(off, csz)], i_d)
            cp1 = pltpu.async_copy(ts_hbm.at[i_s], r_s, sem1)
            cp2 = pltpu.async_copy(td_hbm.at[i_d], r_d, sem2)
            cp1.wait()
            cp2.wait()
            pltpu.sync_copy(r_s, gs_hbm.at[pl.ds(off, csz)])
            pltpu.sync_copy(r_d, gd_hbm.at[pl.ds(off, csz)])

        def body(j, carry):
            chunk(base + j * c, c)
            return carry

        lax.fori_loop(0, n_chunks, body, 0)
        if rem:
            chunk(base + n_chunks * c, rem)

    return gk(table_s, table_d, edge_src, edge_dst)


def _sc_scatter(emsg, edge_dst, n):
    e, d = emsg.shape
    half = n // 2
    per_s = e // 16
    c = 128
    n_chunks = per_s // c
    rem = per_s - n_chunks * c
    mesh = plsc.VectorSubcoreMesh(core_axis_name="c", subcore_axis_name="s")

    @functools.partial(
        pl.kernel, mesh=mesh,
        out_type=jax.ShapeDtypeStruct((n, d), F32),
        scratch_types=[pltpu.VMEM((c,), jnp.int32), pltpu.VMEM((c, d), F32),
                       pltpu.VMEM((rem,), jnp.int32), pltpu.VMEM((rem, d), F32)],
    )
    def sk(emsg_hbm, dst_hbm, out_hbm, idx_v, rows_v, idx_r, rows_r):
        cid = lax.axis_index("c")
        sid = lax.axis_index("s")
        lo = cid * half

        def zrow(i, carry):
            def zcol(j, carry2):
                rows_v[i, pl.ds(j * 16, 16)] = jnp.zeros((16,), F32)
                return carry2
            return lax.fori_loop(0, d // 16, zcol, carry)

        lax.fori_loop(0, c, zrow, 0)

        def zacc(k, carry):
            blk = sid + 16 * k

            @pl.when((blk + 1) * c <= half)
            def _():
                pltpu.sync_copy(rows_v, out_hbm.at[pl.ds(lo + blk * c, c)])

            return carry

        lax.fori_loop(0, 3, zacc, 0)

        @pl.when(sid == 0)
        def _():
            pltpu.sync_copy(rows_v.at[pl.ds(0, 8)],
                            out_hbm.at[pl.ds(lo + 4992, 8)])

        plsc.subcore_barrier()

        def chunk(off, csz, idx, rows):
            pltpu.sync_copy(dst_hbm.at[pl.ds(off, csz)], idx)
            pltpu.sync_copy(emsg_hbm.at[pl.ds(off, csz)], rows)
            for i in range(csz // 16):
                dv = idx[pl.ds(i * 16, 16)]
                ok = (dv >= lo) & (dv < lo + half)
                idx[pl.ds(i * 16, 16)] = jnp.where(ok, dv, -1)
            pltpu.sync_copy(rows, out_hbm.at[plsc.Indices(idx, ignored_value=-1)],
                            add=True)

        def body(j, carry):
            chunk(sid * per_s + j * c, c, idx_v, rows_v)
            return carry

        lax.fori_loop(0, n_chunks, body, 0)
        if rem:
            chunk(sid * per_s + n_chunks * c, rem, idx_r, rows_r)

    return sk(emsg, edge_dst)


def _node_out_body(nmsg_ref, ns_ref, linw, linb, out_ref):
    out_ref[...] = _dot(nmsg_ref[...], linw[...]) + linb[...] + ns_ref[...]


def _node_out(nmsg, ns, p, bn=1000):
    n, dn = ns.shape
    grid = n // bn

    def row_spec(d):
        return pl.BlockSpec((bn, d), lambda i: (i, 0))

    def w_spec(a):
        return pl.BlockSpec(a.shape, lambda i: tuple(0 for _ in a.shape))

    linb = p["lin_b"].reshape(1, -1)
    return pl.pallas_call(
        _node_out_body,
        grid=(grid,),
        in_specs=[row_spec(256), row_spec(dn), w_spec(p["lin_w"]), w_spec(linb)],
        out_specs=row_spec(dn),
        out_shape=jax.ShapeDtypeStruct((n, dn), F32),
    )(nmsg, ns, p["lin_w"], linb)


def kernel(node_fea, edge_fea, edge_sh, edge_length_embedding, edge_vec, D,
           params, edge_src, edge_dst, batch):
    p = params
    n = node_fea.shape[0]
    e = edge_fea.shape[0]
    table_s, table_d, ns = _node_prep(node_fea, p)
    gs, gd = _sc_gather(table_s, table_d, edge_src, edge_dst)
    dm = D.reshape(e, 9)
    emsg, edge_out = _edge_stage(edge_fea, edge_length_embedding, dm,
                                 edge_vec, gs, gd, p)
    nmsg = jax.ops.segment_sum(emsg, edge_dst, num_segments=n)
    node_out = _node_out(nmsg, ns, p)
    return node_out, edge_out


# speedup vs baseline: 11.5313x; 1.0003x over previous
import functools
import math

import jax
import jax.numpy as jnp
from jax import lax
from jax.experimental import pallas as pl
from jax.experimental.pallas import tpu as pltpu
from jax.experimental.pallas import tpu_sc as plsc

SC = 128
H = 8
QK = 16
HD = 32
F32 = jnp.float32


def _ln(x, g, b, eps=1e-6):
    m = jnp.mean(x, axis=-1, keepdims=True)
    v = jnp.mean((x - m) ** 2, axis=-1, keepdims=True)
    return (x - m) / jnp.sqrt(v + eps) * g + b


def _silu(x):
    return x * jax.nn.sigmoid(x)


def _dot(a, b):
    return jax.lax.dot(a, b, preferred_element_type=F32)


def _node_prep_body(x_ref, lng, lnb, qw1, qb1, qg, qbn, qw2, qb2,
                    kw1, kb1, kg, kbn, kw2, kb2, ws, wd, nsw, nsb,
                    ts_ref, td_ref, ns_ref):
    x = x_ref[...]
    node = _ln(x, lng[...], lnb[...])
    scal = node[:, :SC]

    def qkproj(w1, b1, g, bn, w2, b2):
        h = _silu(_ln(_dot(scal, w1[...]) + b1[...], g[...], bn[...]))
        return _dot(h, w2[...]) + b2[...]

    qn = qkproj(qw1, qb1, qg, qbn, qw2, qb2)
    kn = qkproj(kw1, kb1, kg, kbn, kw2, kb2)
    ts_ref[:, :256] = _dot(node, ws[...])
    ts_ref[:, 256:] = kn
    td_ref[:, :256] = _dot(node, wd[...])
    td_ref[:, 256:] = qn
    ns_ref[...] = _dot(x, nsw[...]) + nsb[...]


def _node_prep(node_fea, p, bn=1000):
    n, dn = node_fea.shape
    grid = n // bn

    def row_spec(d):
        return pl.BlockSpec((bn, d), lambda i: (i, 0))

    def w_spec(a):
        return pl.BlockSpec(a.shape, lambda i: tuple(0 for _ in a.shape))

    r2 = lambda a: a.reshape(1, -1)
    weights = [r2(p["ln_node_g"]), r2(p["ln_node_b"]),
               p["q_w1"], r2(p["q_b1"]), r2(p["q_g"]), r2(p["q_bn"]),
               p["q_w2"], r2(p["q_b2"]),
               p["k_w1"], r2(p["k_b1"]), r2(p["k_g"]), r2(p["k_bn"]),
               p["k_w2"], r2(p["k_b2"]),
               p["pre_w"][:dn], p["pre_w"][dn:2 * dn],
               p["ns_w"], r2(p["ns_b"])]
    return pl.pallas_call(
        _node_prep_body,
        grid=(grid,),
        in_specs=[row_spec(dn)] + [w_spec(a) for a in weights],
        out_specs=[row_spec(384), row_spec(384), row_spec(dn)],
        out_shape=[jax.ShapeDtypeStruct((n, 384), F32),
                   jax.ShapeDtypeStruct((n, 384), F32),
                   jax.ShapeDtypeStruct((n, dn), F32)],
    )(node_fea, *weights)


def _edge_body(ef_ref, el_ref, dm_ref, ev_ref, gs_ref, gd_ref,
               lneg, lneb, aw1, ab1, ag1, abg1, aw2, ab2, ag2, abg2, aw3, ab3,
               we, preb, c1r1, c1r2, c1w, c1d, c1g, c1b,
               c2r1, c2r2, c2w, c2d, onw, onb, linew, lineb, esw, esb,
               sel, expm, emsg_ref, eout_ref):
    ef = ef_ref[...]
    el = el_ref[...]
    edge = _ln(ef, lneg[...], lneb[...])
    h = _silu(_ln(_dot(el, aw1[...]) + ab1[...], ag1[...], abg1[...]))
    h = _silu(_ln(_dot(h, aw2[...]) + ab2[...], ag2[...], abg2[...]))
    bias = _dot(h, aw3[...]) + ab3[...]
    gs = gs_ref[...]
    gd = gd_ref[...]
    qk = _dot(gs[:, 256:] * gd[:, 256:], sel[...]) * (1.0 / math.sqrt(QK))
    alpha = qk + bias
    msg = gs[:, :256] + gd[:, :256] + _dot(edge, we[...]) + preb[...]
    dm = dm_ref[...]
    r1 = _dot(_silu(_dot(el, c1r1[...])), c1r2[...])
    v = _dot(msg * r1, c1w[...]) + _dot(dm, c1d[...])
    v = _silu(_ln(v, c1g[...], c1b[...]))
    r2 = _dot(_silu(_dot(el, c2r1[...])), c2r2[...])
    value = _dot(v * r2, c2w[...]) + _dot(dm, c2d[...])
    av = _dot(alpha, expm[...])
    emsg_ref[...] = value * av
    ev = ev_ref[...]
    ons = (ev[:, 0:1] * ev[:, 0:1] + ev[:, 1:2] * ev[:, 1:2]
           + ev[:, 2:3] * ev[:, 2:3]) < 1e-20

    @pl.when(jnp.any(ons))
    def _():
        onsite_val = _dot(msg, onw[...]) + onb[...]
        emsg_ref[...] = jnp.where(ons, onsite_val, value) * av

    eout_ref[...] = (_dot(emsg_ref[...], linew[...]) + lineb[...]
                     + _dot(ef, esw[...]) + esb[...])


def _edge_stage(edge_fea, elen, dm, edge_vec, gs, gd, p, be=1280):
    e, de = edge_fea.shape
    grid = e // be
    sel = (jnp.arange(SC)[:, None] // QK == jnp.arange(H)[None, :]).astype(F32)
    expm = (jnp.arange(H)[:, None] == jnp.arange(H * HD)[None, :] // HD).astype(F32)

    def row_spec(d):
        return pl.BlockSpec((be, d), lambda i: (i, 0))

    def w_spec(a):
        return pl.BlockSpec(a.shape, lambda i: tuple(0 for _ in a.shape))

    r2 = lambda a: a.reshape(1, -1)
    weights = [r2(p["ln_edge_g"]), r2(p["ln_edge_b"]),
               p["a_w1"], r2(p["a_b1"]), r2(p["a_g1"]), r2(p["a_bg1"]),
               p["a_w2"], r2(p["a_b2"]), r2(p["a_g2"]), r2(p["a_bg2"]),
               p["a_w3"], r2(p["a_b3"]),
               p["pre_w"][512:], r2(p["pre_b"]),
               p["c1_r1"], p["c1_r2"], p["c1_w"], p["c1_d"],
               r2(p["c1_g"]), r2(p["c1_b"]),
               p["c2_r1"], p["c2_r2"], p["c2_w"], p["c2_d"],
               p["on_w"], r2(p["on_b"]),
               p["line_w"], r2(p["line_b"]), p["es_w"], r2(p["es_b"]),
               sel, expm]
    return pl.pallas_call(
        _edge_body,
        grid=(grid,),
        in_specs=[row_spec(de), row_spec(64), row_spec(9), row_spec(3),
                  row_spec(384), row_spec(384)] + [w_spec(a) for a in weights],
        out_specs=[row_spec(256), row_spec(de)],
        out_shape=[jax.ShapeDtypeStruct((e, 256), F32),
                   jax.ShapeDtypeStruct((e, de), F32)],
    )(edge_fea, elen, dm, edge_vec, gs, gd, *weights)


def _sc_gather(table_s, table_d, edge_src, edge_dst):
    e = edge_src.shape[0]
    d = table_s.shape[1]
    nw = 32
    per_w = e // nw
    c = 128
    n_chunks = per_w // c
    rem = per_w - n_chunks * c
    mesh = plsc.VectorSubcoreMesh(core_axis_name="c", subcore_axis_name="s")

    @functools.partial(
        pl.kernel, mesh=mesh,
        out_type=[jax.ShapeDtypeStruct((e, d), F32),
                  jax.ShapeDtypeStruct((e, d), F32)],
        scratch_types=[pltpu.VMEM((c,), jnp.int32), pltpu.VMEM((c,), jnp.int32),
                       pltpu.VMEM((c, d), F32), pltpu.VMEM((c, d), F32),
                       pltpu.SemaphoreType.DMA, pltpu.SemaphoreType.DMA],
    )
    def gk(ts_hbm, td_hbm, src_hbm, dst_hbm, gs_hbm, gd_hbm,
           idx_s, idx_d, rows_s, rows_d, sem1, sem2):
        wid = lax.axis_index("s") * 2 + lax.axis_index("c")
        base = wid * per_w

        def chunk(off, csz):
            i_s = idx_s.at[pl.ds(0, csz)] if csz != c else idx_s
            i_d = idx_d.at[pl.ds(0, csz)] if csz != c else idx_d
            r_s = rows_s.at[pl.ds(0, csz)] if csz != c else rows_s
            r_d = rows_d.at[pl.ds(0, csz)] if csz != c else rows_d
            pltpu.sync_copy(src_hbm.at[pl.ds(off, csz)], i_s)
            pltpu.sync_copy(dst_hbm.at[pl.ds(off, csz)], i_d)
            cp1 = pltpu.async_copy(ts_hbm.at[i_s], r_s, sem1)
            cp2 = pltpu.async_copy(td_hbm.at[i_d], r_d, sem2)
            cp1.wait()
            cp2.wait()
            pltpu.sync_copy(r_s, gs_hbm.at[pl.ds(off, csz)])
            pltpu.sync_copy(r_d, gd_hbm.at[pl.ds(off, csz)])

        def body(j, carry):
            chunk(base + j * c, c)
            return carry

        lax.fori_loop(0, n_chunks, body, 0)
        if rem:
            chunk(base + n_chunks * c, rem)

    return gk(table_s, table_d, edge_src, edge_dst)


def _sc_scatter(emsg, edge_dst, n):
    e, d = emsg.shape
    half = n // 2
    acc_rows = 5248
    per_s = e // 16
    c = 128
    n_chunks = per_s // c
    rem = per_s - n_chunks * c
    mesh = plsc.VectorSubcoreMesh(core_axis_name="c", subcore_axis_name="s")

    @functools.partial(
        pl.kernel, mesh=mesh,
        out_type=jax.ShapeDtypeStruct((n, d), F32),
        scratch_types=[pltpu.VMEM((c,), jnp.int32), pltpu.VMEM((c, d), F32),
                       pltpu.VMEM((rem,), jnp.int32),
                       pltpu.VMEM_SHARED((acc_rows, d), F32)],
    )
    def sk(emsg_hbm, dst_hbm, out_hbm, idx_v, rows_v, idx_r, acc):
        cid = lax.axis_index("c")
        sid = lax.axis_index("s")
        lo = cid * half

        def zrow(i, carry):
            def zcol(j, carry2):
                rows_v[i, pl.ds(j * 16, 16)] = jnp.zeros((16,), F32)
                return carry2
            return lax.fori_loop(0, d // 16, zcol, carry)

        lax.fori_loop(0, c, zrow, 0)

        def zacc(k, carry):
            blk = sid + 16 * k

            @pl.when(blk < acc_rows // c)
            def _():
                pltpu.sync_copy(rows_v, acc.at[pl.ds(blk * c, c)])

            return carry

        lax.fori_loop(0, 3, zacc, 0)
        plsc.subcore_barrier()

        def chunk(off, csz, idx):
            pltpu.sync_copy(dst_hbm.at[pl.ds(off, csz)], idx)
            pltpu.sync_copy(emsg_hbm.at[pl.ds(off, csz)], rows_v.at[pl.ds(0, csz)])
            for i in range(csz // 16):
                dv = idx[pl.ds(i * 16, 16)]
                local = dv - lo
                ok = (local >= 0) & (local < half)
                idx[pl.ds(i * 16, 16)] = jnp.where(ok, local, -1)
            pltpu.sync_copy(rows_v.at[pl.ds(0, csz)],
                            acc.at[plsc.Indices(idx, ignored_value=-1)],
                            add=True)

        def body(j, carry):
            chunk(sid * per_s + j * c, c, idx_v)
            return carry

        lax.fori_loop(0, n_chunks, body, 0)
        if rem:
            chunk(sid * per_s + n_chunks * c, rem, idx_r)
        plsc.subcore_barrier()

        pltpu.sync_copy(acc.at[pl.ds(sid * 312, 312)],
                        out_hbm.at[pl.ds(lo + sid * 312, 312)])

        @pl.when(sid == 0)
        def _():
            pltpu.sync_copy(acc.at[pl.ds(4992, 8)],
                            out_hbm.at[pl.ds(lo + 4992, 8)])

    return sk(emsg, edge_dst)


def _node_out_body(nmsg_ref, ns_ref, linw, linb, out_ref):
    out_ref[...] = _dot(nmsg_ref[...], linw[...]) + linb[...] + ns_ref[...]


def _node_out(nmsg, ns, p, bn=1000):
    n, dn = ns.shape
    grid = n // bn

    def row_spec(d):
        return pl.BlockSpec((bn, d), lambda i: (i, 0))

    def w_spec(a):
        return pl.BlockSpec(a.shape, lambda i: tuple(0 for _ in a.shape))

    linb = p["lin_b"].reshape(1, -1)
    return pl.pallas_call(
        _node_out_body,
        grid=(grid,),
        in_specs=[row_spec(256), row_spec(dn), w_spec(p["lin_w"]), w_spec(linb)],
        out_specs=row_spec(dn),
        out_shape=jax.ShapeDtypeStruct((n, dn), F32),
    )(nmsg, ns, p["lin_w"], linb)


def kernel(node_fea, edge_fea, edge_sh, edge_length_embedding, edge_vec, D,
           params, edge_src, edge_dst, batch):
    p = params
    n = node_fea.shape[0]
    e = edge_fea.shape[0]
    table_s, table_d, ns = _node_prep(node_fea, p)
    gs, gd = _sc_gather(table_s, table_d, edge_src, edge_dst)
    dm = D.reshape(e, 9)
    emsg, edge_out = _edge_stage(edge_fea, edge_length_embedding, dm,
                                 edge_vec, gs, gd, p)
    nmsg = jax.ops.segment_sum(emsg, edge_dst, num_segments=n)
    node_out = _node_out(nmsg, ns, p)
    return node_out, edge_out
